# Initial kernel scaffold; baseline (speedup 1.0000x reference)
import functools

import jax
import jax.numpy as jnp
from jax import lax
from jax.experimental import pallas as pl
from jax.experimental.pallas import tpu as pltpu
from jax.experimental.pallas import tpu_sc as plsc

V = 100000
H = 8
D = 16
ROW = H * (D + 1)
OUT = H * D
B = 16384
NC = 2
NS = 16
NW = NC * NS
BPW = B // NW
CH = 128
NCH = BPW // CH

_mesh = plsc.VectorSubcoreMesh(core_axis_name="c", subcore_axis_name="s")


@functools.partial(
    pl.kernel,
    mesh=_mesh,
    out_type=jax.ShapeDtypeStruct((B, OUT), jnp.float32),
    scratch_types=[
        pltpu.VMEM((NCH, CH), jnp.int32),
        pltpu.VMEM((CH, ROW), jnp.float32),
        pltpu.VMEM((CH, OUT), jnp.float32),
        pltpu.SemaphoreType.DMA,
    ],
)
def _sc_embed(table_hbm, idx_hbm, out_hbm, idx_v, gbuf, obuf, gsem):
    wid = lax.axis_index("s") * NC + lax.axis_index("c")
    base = wid * BPW
    pltpu.sync_copy(idx_hbm.at[wid], idx_v)

    for c in range(NCH):
        pltpu.async_copy(table_hbm.at[idx_v.at[c]], gbuf, gsem).wait()

        def body(j, _):
            for h in range(H):
                num = gbuf[j, pl.ds(h * (D + 1), D)]
                r = 1.0 / gbuf[j, h * (D + 1) + D]
                obuf[j, pl.ds(h * D, D)] = num * r
            return 0

        lax.fori_loop(0, CH, body, 0)
        pltpu.sync_copy(obuf, out_hbm.at[pl.ds(base + c * CH, CH)])


def kernel(memory, nodes):
    table = memory.reshape(V, ROW)
    idx = nodes.astype(jnp.int32).reshape(NW, NCH, CH)
    return _sc_embed(table, idx)


# speedup vs baseline: 1.1634x; 1.1634x over previous
import functools

import jax
import jax.numpy as jnp
from jax import lax
from jax.experimental import pallas as pl
from jax.experimental.pallas import tpu as pltpu
from jax.experimental.pallas import tpu_sc as plsc

V = 100000
H = 8
D = 16
ROW = H * (D + 1)
OUT = H * D
B = 16384
NC = 2
NS = 16
NW = NC * NS
BPW = B // NW
CH = 128
NCH = BPW // CH

_mesh = plsc.VectorSubcoreMesh(core_axis_name="c", subcore_axis_name="s")


@functools.partial(
    pl.kernel,
    mesh=_mesh,
    out_type=jax.ShapeDtypeStruct((B, OUT), jnp.float32),
    compiler_params=pltpu.CompilerParams(use_tc_tiling_on_sc=False),
    scratch_types=[
        pltpu.VMEM((NCH, CH), jnp.int32),
        pltpu.VMEM((CH, ROW), jnp.float32),
        pltpu.VMEM((CH, OUT), jnp.float32),
        pltpu.SemaphoreType.DMA,
    ],
)
def _sc_embed(table_hbm, idx_hbm, out_hbm, idx_v, gbuf, obuf, gsem):
    wid = lax.axis_index("s") * NC + lax.axis_index("c")
    base = wid * BPW
    pltpu.sync_copy(idx_hbm.at[wid], idx_v)

    for c in range(NCH):
        pltpu.async_copy(table_hbm.at[idx_v.at[c]], gbuf, gsem).wait()

        def body(j, _):
            for h in range(H):
                num = gbuf[j, pl.ds(h * (D + 1), D)]
                shifted = gbuf[j, pl.ds(h * (D + 1) + 1, D)]
                den = jnp.broadcast_to(shifted[D - 1], (D,))
                obuf[j, pl.ds(h * D, D)] = num / den
            return 0

        lax.fori_loop(0, CH, body, 0)
        pltpu.sync_copy(obuf, out_hbm.at[pl.ds(base + c * CH, CH)])


def kernel(memory, nodes):
    table = memory.reshape(V, ROW)
    idx = nodes.astype(jnp.int32).reshape(NW, NCH, CH)
    return _sc_embed(table, idx)


# speedup vs baseline: 2.9376x; 2.5250x over previous
import functools

import jax
import jax.numpy as jnp
from jax import lax
from jax.experimental import pallas as pl
from jax.experimental.pallas import tpu as pltpu
from jax.experimental.pallas import tpu_sc as plsc

V = 100000
H = 8
D = 16
B = 16384
OUT = H * D
NC = 2
NS = 16
NW = NC * NS
BPW = B // NW
G = 16
NG = BPW // G

_mesh = plsc.VectorSubcoreMesh(core_axis_name="c", subcore_axis_name="s")


@functools.partial(
    pl.kernel,
    mesh=_mesh,
    out_type=jax.ShapeDtypeStruct((B, OUT), jnp.float32),
    compiler_params=pltpu.CompilerParams(use_tc_tiling_on_sc=True),
    scratch_types=[
        pltpu.VMEM((BPW,), jnp.int32),
        pltpu.VMEM((G, H, D + 1), jnp.float32),
        pltpu.VMEM((BPW, OUT), jnp.float32),
        pltpu.SemaphoreType.DMA,
    ],
)
def _sc_embed(table_hbm, idx_hbm, out_hbm, idx_v, gbuf, obuf, gsem):
    wid = lax.axis_index("s") * NC + lax.axis_index("c")
    base = wid * BPW
    pltpu.sync_copy(idx_hbm.at[pl.ds(base, BPW)], idx_v)

    def burst(t, _):
        iv = idx_v[pl.ds(t * G, G)]
        copies = [
            pltpu.async_copy(table_hbm.at[iv[l]], gbuf.at[l], gsem)
            for l in range(G)
        ]
        for cp in copies:
            cp.wait()
        for l in range(G):
            for h in range(H):
                num = gbuf[l, h, pl.ds(0, D)]
                shifted = gbuf[l, h, pl.ds(1, D)]
                den = jnp.broadcast_to(shifted[D - 1], (D,))
                obuf[t * G + l, pl.ds(h * D, D)] = num / den
        return 0

    lax.fori_loop(0, NG, burst, 0)
    pltpu.sync_copy(obuf, out_hbm.at[pl.ds(base, BPW)])


def kernel(memory, nodes):
    return _sc_embed(memory, nodes.astype(jnp.int32))
